# baseline (speedup 1.0000x reference)
import functools

import jax
import jax.numpy as jnp
from jax import lax
from jax.experimental import pallas as pl
from jax.experimental.pallas import tpu as pltpu
from jax.experimental.pallas import tpu_sc as plsc

_N = 1024
_D = 256
_K = 4
_BB = 1024

_NC = 2
_NS = 16
_NW = _NC * _NS
_L = 16
_CH = 32


def _rep_body(d_ref, rep_ref):
    rep_ref[0, ...] = d_ref[...]


def _replicate_tc(dictionary):
    return pl.pallas_call(
        _rep_body,
        grid=(_NW,),
        in_specs=[pl.BlockSpec((_N, _D), lambda i: (0, 0))],
        out_specs=pl.BlockSpec((1, _N, _D), lambda i: (i, 0, 0)),
        out_shape=jax.ShapeDtypeStruct((_NW, _N, _D), jnp.float32),
    )(dictionary)


def _tc_body(x_ref, d_ref, idx_ref):
    x = x_ref[...]
    d = d_ref[...]
    xsq = jnp.sum(x * x, axis=1, keepdims=True)
    dsq = jnp.sum(d * d, axis=1)[None, :]
    xd = jax.lax.dot_general(
        x, d, (((1,), (1,)), ((), ())),
        preferred_element_type=jnp.float32)
    work = xsq + dsq - 2.0 * xd

    iota = jax.lax.broadcasted_iota(jnp.int32, (_BB, _N), 1)
    cols = []
    for k in range(_K):
        m = jnp.min(work, axis=1, keepdims=True)
        idx_k = jnp.min(jnp.where(work == m, iota, _N), axis=1)
        work = jnp.where(iota == idx_k[:, None], jnp.inf, work)
        cols.append(idx_k[:, None])
    idx_ref[...] = jnp.concatenate(cols, axis=1)


def _topk_tc(inputs_flatten, dictionary):
    b = inputs_flatten.shape[0]
    return pl.pallas_call(
        _tc_body,
        grid=(b // _BB,),
        in_specs=[
            pl.BlockSpec((_BB, _D), lambda i: (i, 0)),
            pl.BlockSpec((_N, _D), lambda i: (0, 0)),
        ],
        out_specs=pl.BlockSpec((_BB, _K), lambda i: (i, 0)),
        out_shape=jax.ShapeDtypeStruct((b, _K), jnp.int32),
    )(inputs_flatten, dictionary)


def _sc_gather_avg(idx_flat, table_rep, b):
    rw = b // _NW
    nchunks = rw // _CH
    mesh = plsc.VectorSubcoreMesh(core_axis_name="c", subcore_axis_name="s")

    @functools.partial(
        pl.kernel,
        mesh=mesh,
        out_type=jax.ShapeDtypeStruct((b, _D), jnp.float32),
        scratch_types=[
            pltpu.VMEM((rw * _K,), jnp.int32),
            pltpu.VMEM((3, _K * _CH, _D), jnp.float32),
            pltpu.VMEM((2, _CH, _D), jnp.float32),
            pltpu.SemaphoreType.DMA,
            pltpu.SemaphoreType.DMA,
            pltpu.SemaphoreType.DMA,
            pltpu.SemaphoreType.DMA,
            pltpu.SemaphoreType.DMA,
        ],
    )
    def k(idx_hbm, table_hbm, out_hbm, idx_v, rows_v, out_v,
          gsem0, gsem1, gsem2, osem0, osem1):
        wid = lax.axis_index("s") * _NC + lax.axis_index("c")
        base = wid * rw
        pltpu.sync_copy(idx_hbm.at[pl.ds(base * _K, rw * _K)], idx_v)
        off = wid * _N
        for i in range(rw * _K // _L):
            sl = pl.ds(i * _L, _L)
            idx_v[sl] = idx_v[sl] + off
        gsems = (gsem0, gsem1, gsem2)
        osems = (osem0, osem1)

        def start_gather(c):
            return pltpu.async_copy(
                table_hbm.at[idx_v.at[pl.ds(c * _K * _CH, _K * _CH)]],
                rows_v.at[c % 3], gsems[c % 3])

        g_pend = [start_gather(0), start_gather(1)]
        out_pend = [None, None]
        for c in range(nchunks):
            g_pend.pop(0).wait()
            if c + 2 < nchunks:
                g_pend.append(start_gather(c + 2))
            rv = rows_v.at[c % 3]
            ov = out_v.at[c % 2]
            if out_pend[c % 2] is not None:
                out_pend[c % 2].wait()

            def row(r, _, rv=rv, ov=ov):
                for j in range(_D // _L):
                    sl = pl.ds(j * _L, _L)
                    acc = (rv[r * _K, sl] + rv[r * _K + 1, sl]
                           + rv[r * _K + 2, sl] + rv[r * _K + 3, sl])
                    ov[r, sl] = acc * 0.25
                return 0

            lax.fori_loop(0, _CH, row, 0, unroll=False)
            out_pend[c % 2] = pltpu.async_copy(
                ov, out_hbm.at[pl.ds(base + c * _CH, _CH)], osems[c % 2])
        for p in out_pend:
            if p is not None:
                p.wait()

    return k(idx_flat, table_rep.reshape(_NW * _N, _D))


_SLICES = 2


def kernel(inputs_flatten, dictionary):
    b = inputs_flatten.shape[0]
    bs = b // _SLICES
    table_rep = _replicate_tc(dictionary)
    idxs = [
        _topk_tc(inputs_flatten[s * bs:(s + 1) * bs], dictionary)
        for s in range(_SLICES)
    ]
    embs = [
        _sc_gather_avg(idx_s.reshape(bs * _K), table_rep, bs)
        for idx_s in idxs
    ]
    return (jnp.concatenate(embs, axis=0), jnp.concatenate(idxs, axis=0))


# speedup vs baseline: 1.0592x; 1.0592x over previous
import functools

import jax
import jax.numpy as jnp
from jax import lax
from jax.experimental import pallas as pl
from jax.experimental.pallas import tpu as pltpu
from jax.experimental.pallas import tpu_sc as plsc

_N = 1024
_D = 256
_K = 4
_BB = 1024

_NC = 2
_NS = 16
_NW = _NC * _NS
_L = 16
_CH = 32


def _tc_body(x_ref, d_ref, idx_ref, rep_ref):
    x = x_ref[...]
    d = d_ref[...]
    rep_ref[0, ...] = d
    rep_ref[1, ...] = d
    xsq = jnp.sum(x * x, axis=1, keepdims=True)
    dsq = jnp.sum(d * d, axis=1)[None, :]
    xd = jax.lax.dot_general(
        x, d, (((1,), (1,)), ((), ())),
        preferred_element_type=jnp.float32)
    work = xsq + dsq - 2.0 * xd

    iota = jax.lax.broadcasted_iota(jnp.int32, (_BB, _N), 1)
    cols = []
    for k in range(_K):
        m = jnp.min(work, axis=1, keepdims=True)
        idx_k = jnp.min(jnp.where(work == m, iota, _N), axis=1)
        work = jnp.where(iota == idx_k[:, None], jnp.inf, work)
        cols.append(idx_k[:, None])
    idx_ref[...] = jnp.concatenate(cols, axis=1)


def _topk_tc(inputs_flatten, dictionary):
    b = inputs_flatten.shape[0]
    return pl.pallas_call(
        _tc_body,
        grid=(b // _BB,),
        in_specs=[
            pl.BlockSpec((_BB, _D), lambda i: (i, 0)),
            pl.BlockSpec((_N, _D), lambda i: (0, 0)),
        ],
        out_specs=[
            pl.BlockSpec((_BB, _K), lambda i: (i, 0)),
            pl.BlockSpec((2, _N, _D), lambda i: (i, 0, 0)),
        ],
        out_shape=[
            jax.ShapeDtypeStruct((b, _K), jnp.int32),
            jax.ShapeDtypeStruct((_NW, _N, _D), jnp.float32),
        ],
    )(inputs_flatten, dictionary)


def _sc_gather_avg(idx_flat, table_rep, b):
    rw = b // _NW
    nchunks = rw // _CH
    mesh = plsc.VectorSubcoreMesh(core_axis_name="c", subcore_axis_name="s")

    @functools.partial(
        pl.kernel,
        mesh=mesh,
        out_type=jax.ShapeDtypeStruct((b, _D), jnp.float32),
        scratch_types=[
            pltpu.VMEM((rw * _K,), jnp.int32),
            pltpu.VMEM((3, _K * _CH, _D), jnp.float32),
            pltpu.VMEM((2, _CH, _D), jnp.float32),
            pltpu.SemaphoreType.DMA,
            pltpu.SemaphoreType.DMA,
            pltpu.SemaphoreType.DMA,
            pltpu.SemaphoreType.DMA,
            pltpu.SemaphoreType.DMA,
        ],
    )
    def k(idx_hbm, table_hbm, out_hbm, idx_v, rows_v, out_v,
          gsem0, gsem1, gsem2, osem0, osem1):
        wid = lax.axis_index("s") * _NC + lax.axis_index("c")
        base = wid * rw
        pltpu.sync_copy(idx_hbm.at[pl.ds(base * _K, rw * _K)], idx_v)
        off = wid * _N
        for i in range(rw * _K // _L):
            sl = pl.ds(i * _L, _L)
            idx_v[sl] = idx_v[sl] + off
        gsems = (gsem0, gsem1, gsem2)
        osems = (osem0, osem1)

        def start_gather(c):
            return pltpu.async_copy(
                table_hbm.at[idx_v.at[pl.ds(c * _K * _CH, _K * _CH)]],
                rows_v.at[c % 3], gsems[c % 3])

        g_pend = [start_gather(0), start_gather(1)]
        out_pend = [None, None]
        for c in range(nchunks):
            g_pend.pop(0).wait()
            if c + 2 < nchunks:
                g_pend.append(start_gather(c + 2))
            rv = rows_v.at[c % 3]
            ov = out_v.at[c % 2]
            if out_pend[c % 2] is not None:
                out_pend[c % 2].wait()

            def row(r, _, rv=rv, ov=ov):
                for j in range(_D // _L):
                    sl = pl.ds(j * _L, _L)
                    acc = (rv[r * _K, sl] + rv[r * _K + 1, sl]
                           + rv[r * _K + 2, sl] + rv[r * _K + 3, sl])
                    ov[r, sl] = acc * 0.25
                return 0

            lax.fori_loop(0, _CH, row, 0, unroll=False)
            out_pend[c % 2] = pltpu.async_copy(
                ov, out_hbm.at[pl.ds(base + c * _CH, _CH)], osems[c % 2])
        for p in out_pend:
            if p is not None:
                p.wait()

    return k(idx_flat, table_rep.reshape(_NW * _N, _D))


def kernel(inputs_flatten, dictionary):
    b = inputs_flatten.shape[0]
    idx, table_rep = _topk_tc(inputs_flatten, dictionary)
    emb = _sc_gather_avg(idx.reshape(b * _K), table_rep, b)
    return (emb, idx)
